# baseline (speedup 1.0000x reference)
import jax
import jax.numpy as jnp
from jax import lax
from jax.experimental import pallas as pl
from jax.experimental.pallas import tpu as pltpu
from jax.experimental.pallas import tpu_sc as plsc

B_ = 4096
C_ = 1000
L_ = 50
NC_ = 2
NS_ = 16
NW_ = NC_ * NS_
BPW_ = B_ // NW_
EPW_ = BPW_ * L_
SLAB_ = C_ * L_
CHUNK_ = 128
NCHUNK_ = EPW_ // CHUNK_
FBLK_ = 8


def _tc_fill(o_ref):
    o_ref[...] = jnp.zeros((FBLK_, SLAB_), jnp.int32)


def _sc_scatter(tgt_hbm, out_hbm, tgt_v, idx_v, ones_v, sem):
    wid = lax.axis_index("s") * NC_ + lax.axis_index("c")
    base_b = wid * BPW_
    base_e = wid * EPW_

    for c in range(CHUNK_ // 16):
        ones_v[pl.ds(c * 16, 16)] = jnp.ones((16,), jnp.int32)

    pltpu.sync_copy(tgt_hbm.at[pl.ds(base_e, EPW_)], tgt_v)

    lanes = lax.iota(jnp.int32, 16)

    def ibody(j, _):
        for c in range(CHUNK_ // 16):
            k = j * CHUNK_ + c * 16 + lanes
            bl = lax.div(k, L_)
            l = k - bl * L_
            t = tgt_v[pl.ds(j * CHUNK_ + c * 16, 16)]
            idx_v[j, pl.ds(c * 16, 16)] = (base_b + bl) * SLAB_ + t * L_ + l
        return 0
    lax.fori_loop(0, NCHUNK_, ibody, 0)

    def sbody(j, _):
        pltpu.make_async_copy(ones_v, out_hbm.at[idx_v.at[j]], sem).start()
        return 0
    lax.fori_loop(0, NCHUNK_, sbody, 0)

    def sdrain(j, _):
        pltpu.make_async_copy(ones_v, out_hbm.at[idx_v.at[j]], sem).wait()
        return 0
    lax.fori_loop(0, NCHUNK_, sdrain, 0)


_sc_scatter_call = pl.kernel(
    _sc_scatter,
    out_type=(),
    mesh=plsc.VectorSubcoreMesh(core_axis_name="c", subcore_axis_name="s"),
    scratch_types=[
        pltpu.VMEM((EPW_,), jnp.int32),
        pltpu.VMEM((NCHUNK_, CHUNK_), jnp.int32),
        pltpu.VMEM((CHUNK_,), jnp.int32),
        pltpu.SemaphoreType.DMA,
    ],
)


@jax.jit
def kernel(target):
    zeros2d = pl.pallas_call(
        _tc_fill,
        grid=(B_ // FBLK_,),
        out_specs=pl.BlockSpec((FBLK_, SLAB_), lambda i: (i, 0)),
        out_shape=jax.ShapeDtypeStruct((B_, SLAB_), jnp.int32),
    )()
    canvas = jax.new_ref(jnp.reshape(zeros2d, (B_ * SLAB_,)))
    _sc_scatter_call(jnp.reshape(target, (B_ * L_,)), canvas)
    return jnp.reshape(canvas[...], (B_, C_, L_))


# speedup vs baseline: 1.3113x; 1.3113x over previous
import jax
import jax.numpy as jnp
from jax import lax
from jax.experimental import pallas as pl
from jax.experimental.pallas import tpu as pltpu
from jax.experimental.pallas import tpu_sc as plsc

B_ = 4096
C_ = 1000
L_ = 50
N_ = B_ * C_ * L_
NC_ = 2
NS_ = 16
NW_ = NC_ * NS_
BPW_ = B_ // NW_
EPW_ = BPW_ * L_
SLAB_ = C_ * L_
CHUNK_ = 128
NCHUNK_ = EPW_ // CHUNK_
FCH_ = 400000
NFILL_ = N_ // FCH_
QD_ = 8


def _tc_fill(out_ref, zbuf, sems):
    zbuf[...] = jnp.zeros((FCH_,), jnp.int32)

    def fire(i, q):
        pltpu.make_async_copy(zbuf, out_ref.at[pl.ds(i * FCH_, FCH_)],
                              sems.at[q]).start()

    def wait(i, q):
        pltpu.make_async_copy(zbuf, out_ref.at[pl.ds(i * FCH_, FCH_)],
                              sems.at[q]).wait()

    def body(i, _):
        fire(i, lax.rem(i, QD_))

        @pl.when(i >= QD_ - 1)
        def _():
            wait(i, lax.rem(i + 1, QD_))
        return 0
    lax.fori_loop(0, NFILL_, body, 0)

    def drain(j, _):
        wait(0, lax.rem(NFILL_ + 1 + j, QD_))
        return 0
    lax.fori_loop(0, QD_ - 1, drain, 0)


def _sc_scatter(tgt_hbm, out_ref, tgt_v, idx_v, ones_v, sem):
    wid = lax.axis_index("s") * NC_ + lax.axis_index("c")
    base_b = wid * BPW_
    base_e = wid * EPW_

    def obody(j, _):
        for c in range(CHUNK_ // 16):
            ones_v[j, pl.ds(c * 16, 16)] = jnp.ones((16,), jnp.int32)
        return 0
    lax.fori_loop(0, NCHUNK_, obody, 0)

    pltpu.sync_copy(tgt_hbm.at[pl.ds(base_e, EPW_)], tgt_v)

    lanes = lax.iota(jnp.int32, 16)

    def ibody(j, _):
        for c in range(CHUNK_ // 16):
            k = j * CHUNK_ + c * 16 + lanes
            bl = lax.div(k, L_)
            l = k - bl * L_
            t = tgt_v[pl.ds(j * CHUNK_ + c * 16, 16)]
            idx_v[j, pl.ds(c * 16, 16)] = (base_b + bl) * SLAB_ + t * L_ + l
        return 0
    lax.fori_loop(0, NCHUNK_, ibody, 0)

    def sbody(j, _):
        pltpu.make_async_copy(ones_v.at[j], out_ref.at[idx_v.at[j]],
                              sem).start()
        return 0
    lax.fori_loop(0, NCHUNK_, sbody, 0)

    def sdrain(j, _):
        pltpu.make_async_copy(ones_v.at[j], out_ref.at[idx_v.at[j]],
                              sem).wait()
        return 0
    lax.fori_loop(0, NCHUNK_, sdrain, 0)


_tc_fill_call = pl.kernel(
    _tc_fill,
    out_type=(),
    mesh=pltpu.create_tensorcore_mesh("x"),
    scratch_types=[
        pltpu.VMEM((FCH_,), jnp.int32),
        pltpu.SemaphoreType.DMA((QD_,)),
    ],
)

_sc_scatter_call = pl.kernel(
    _sc_scatter,
    out_type=(),
    mesh=plsc.VectorSubcoreMesh(core_axis_name="c", subcore_axis_name="s"),
    scratch_types=[
        pltpu.VMEM((EPW_,), jnp.int32),
        pltpu.VMEM((NCHUNK_, CHUNK_), jnp.int32),
        pltpu.VMEM((NCHUNK_, CHUNK_), jnp.int32),
        pltpu.SemaphoreType.DMA,
    ],
)


@jax.jit
def kernel(target):
    canvas = jax.new_ref(pl.empty((N_,), jnp.int32))
    _tc_fill_call(canvas)
    _sc_scatter_call(jnp.reshape(target, (B_ * L_,)), canvas)
    return jnp.reshape(canvas[...], (B_, C_, L_))
